# baseline (speedup 1.0000x reference)
import functools

import jax
import jax.numpy as jnp
from jax import lax
from jax.experimental import pallas as pl
from jax.experimental.pallas import tpu as pltpu
from jax.experimental.pallas import tpu_sc as plsc

_NC = 2
_NS = 16
_NW = _NC * _NS
_CB = 1024
_NB = 3


def _sqrt16(s):
    i = lax.bitcast_convert_type(s, jnp.int32)
    i = 0x5F3759DF - lax.shift_right_arithmetic(i, 1)
    y = lax.bitcast_convert_type(i, jnp.float32)
    y = y * (1.5 - 0.5 * s * y * y)
    y = y * (1.5 - 0.5 * s * y * y)
    return s * y


def _make_norm_kernel(E):
    nchunk = E // _CB
    mesh = plsc.VectorSubcoreMesh(core_axis_name="c", subcore_axis_name="s")

    @functools.partial(
        pl.kernel,
        mesh=mesh,
        out_type=[
            jax.ShapeDtypeStruct((E,), jnp.float32),
            jax.ShapeDtypeStruct((3, E), jnp.float32),
        ],
        scratch_types=(
            [pltpu.VMEM((3, _CB), jnp.float32)] * _NB
            + [pltpu.VMEM((_CB,), jnp.float32)] * _NB
            + [pltpu.SemaphoreType.DMA] * (3 * _NB)
        ),
    )
    def norm_k(evt_hbm, w_hbm, evtc_hbm, *bufs):
        vbs = bufs[0:_NB]
        obs = bufs[_NB:2 * _NB]
        sems = bufs[2 * _NB:]
        sivs = sems[0:_NB]
        sovs = sems[_NB:2 * _NB]
        sows = sems[2 * _NB:3 * _NB]

        wid = lax.axis_index("s") * _NC + lax.axis_index("c")
        base, rem = nchunk // _NW, nchunk % _NW
        nk = base + jnp.where(wid < rem, 1, 0)
        start = wid * base + lax.min(wid, rem)

        def off_of(i):
            return (start + i) * _CB

        def in_v(i, b):
            return pltpu.make_async_copy(
                evt_hbm.at[:, pl.ds(off_of(i), _CB)], vbs[b], sivs[b])

        def out_v(i, b):
            return pltpu.make_async_copy(
                vbs[b], evtc_hbm.at[:, pl.ds(off_of(i), _CB)], sovs[b])

        def out_w(i, b):
            return pltpu.make_async_copy(
                obs[b], w_hbm.at[pl.ds(off_of(i), _CB)], sows[b])

        for b in range(2):
            @pl.when(nk > b)
            def _(b=b):
                in_v(b, b).start()

        def run_chunk(i, b):
            bn = (b + 2) % _NB

            @pl.when(i + 2 < nk)
            def _():
                @pl.when(i >= 1)
                def _():
                    out_v(i - 1, bn).wait()

                in_v(i + 2, bn).start()

            vb, ob = vbs[b], obs[b]
            in_v(i, b).wait()
            out_v(i, b).start()

            @pl.when(i >= _NB)
            def _():
                out_w(i - _NB, b).wait()

            @plsc.parallel_loop(0, _CB // 16, unroll=8)
            def _(g):
                x = vb[0, pl.ds(g * 16, 16)]
                y = vb[1, pl.ds(g * 16, 16)]
                z = vb[2, pl.ds(g * 16, 16)]
                ob[pl.ds(g * 16, 16)] = _sqrt16(x * x + y * y + z * z)

            out_w(i, b).start()

        def triple(p, c):
            for b in range(_NB):
                i = _NB * p + b

                @pl.when(i < nk)
                def _(i=i, b=b):
                    run_chunk(i, b)

            return c

        nk_max = (nchunk + _NW - 1) // _NW
        lax.fori_loop(0, (nk_max + _NB - 1) // _NB, triple, 0)

        for b in range(_NB):
            i_b = nk - 1 - lax.rem(nk - 1 - b, _NB)

            @pl.when(nk > b)
            def _(i_b=i_b, b=b):
                out_w(i_b, b).wait()
                out_v(i_b, b).wait()

    return norm_k


def kernel(edge_index, edge_vec):
    E = edge_vec.shape[0]
    w, evt_c = _make_norm_kernel(E)(edge_vec.T)
    return (edge_index, w, evt_c.T)


# speedup vs baseline: 1.0928x; 1.0928x over previous
import functools

import jax
import jax.numpy as jnp
from jax import lax
from jax.experimental import pallas as pl
from jax.experimental.pallas import tpu as pltpu
from jax.experimental.pallas import tpu_sc as plsc

_NC = 2
_NS = 16
_NW = _NC * _NS
_CB = 5120
_NB = 3


def _sqrt16(s):
    i = lax.bitcast_convert_type(s, jnp.int32)
    i = 0x5F3759DF - lax.shift_right_arithmetic(i, 1)
    y = lax.bitcast_convert_type(i, jnp.float32)
    y = y * (1.5 - 0.5 * s * y * y)
    y = y * (1.5 - 0.5 * s * y * y)
    return s * y


def _make_norm_kernel(E):
    nchunk = E // _CB
    mesh = plsc.VectorSubcoreMesh(core_axis_name="c", subcore_axis_name="s")

    @functools.partial(
        pl.kernel,
        mesh=mesh,
        out_type=[
            jax.ShapeDtypeStruct((E,), jnp.float32),
            jax.ShapeDtypeStruct((3, E), jnp.float32),
        ],
        scratch_types=(
            [pltpu.VMEM((3, _CB), jnp.float32)] * _NB
            + [pltpu.VMEM((_CB,), jnp.float32)] * _NB
            + [pltpu.SemaphoreType.DMA] * (3 * _NB)
        ),
    )
    def norm_k(evt_hbm, w_hbm, evtc_hbm, *bufs):
        vbs = bufs[0:_NB]
        obs = bufs[_NB:2 * _NB]
        sems = bufs[2 * _NB:]
        sivs = sems[0:_NB]
        sovs = sems[_NB:2 * _NB]
        sows = sems[2 * _NB:3 * _NB]

        wid = lax.axis_index("s") * _NC + lax.axis_index("c")
        base, rem = nchunk // _NW, nchunk % _NW
        nk = base + jnp.where(wid < rem, 1, 0)
        start = wid * base + lax.min(wid, rem)

        def off_of(i):
            return (start + i) * _CB

        def in_v(i, b):
            return pltpu.make_async_copy(
                evt_hbm.at[:, pl.ds(off_of(i), _CB)], vbs[b], sivs[b])

        def out_v(i, b):
            return pltpu.make_async_copy(
                vbs[b], evtc_hbm.at[:, pl.ds(off_of(i), _CB)], sovs[b])

        def out_w(i, b):
            return pltpu.make_async_copy(
                obs[b], w_hbm.at[pl.ds(off_of(i), _CB)], sows[b])

        for b in range(2):
            @pl.when(nk > b)
            def _(b=b):
                in_v(b, b).start()

        def run_chunk(i, b):
            bn = (b + 2) % _NB

            @pl.when(i + 2 < nk)
            def _():
                @pl.when(i >= 1)
                def _():
                    out_v(i - 1, bn).wait()

                in_v(i + 2, bn).start()

            vb, ob = vbs[b], obs[b]
            in_v(i, b).wait()
            out_v(i, b).start()

            @pl.when(i >= _NB)
            def _():
                out_w(i - _NB, b).wait()

            @plsc.parallel_loop(0, _CB // 16, unroll=8)
            def _(g):
                x = vb[0, pl.ds(g * 16, 16)]
                y = vb[1, pl.ds(g * 16, 16)]
                z = vb[2, pl.ds(g * 16, 16)]
                ob[pl.ds(g * 16, 16)] = _sqrt16(x * x + y * y + z * z)

            out_w(i, b).start()

        def triple(p, c):
            for b in range(_NB):
                i = _NB * p + b

                @pl.when(i < nk)
                def _(i=i, b=b):
                    run_chunk(i, b)

            return c

        nk_max = (nchunk + _NW - 1) // _NW
        lax.fori_loop(0, (nk_max + _NB - 1) // _NB, triple, 0)

        for b in range(_NB):
            i_b = nk - 1 - lax.rem(nk - 1 - b, _NB)

            @pl.when(nk > b)
            def _(i_b=i_b, b=b):
                out_w(i_b, b).wait()
                out_v(i_b, b).wait()

    return norm_k


def kernel(edge_index, edge_vec):
    E = edge_vec.shape[0]
    w, evt_c = _make_norm_kernel(E)(edge_vec.T)
    return (edge_index, w, evt_c.T)
